# baseline (speedup 1.0000x reference)
import functools

import jax
import jax.numpy as jnp
from jax import lax
from jax.experimental import pallas as pl
from jax.experimental.pallas import tpu as pltpu
from jax.experimental.pallas import tpu_sc as plsc

NUM_LOCATIONS = 100000
NUM_USERS = 100000
DIM = 64
BATCH = 4096
HIST = 50

NC, NS = 2, 16
NW = NC * NS
B_PER_NW = BATCH // NW

_mesh = plsc.VectorSubcoreMesh(core_axis_name="c", subcore_axis_name="s")


@functools.partial(
    pl.kernel,
    mesh=_mesh,
    out_type=[
        jax.ShapeDtypeStruct((HIST, BATCH // 2, 2 * DIM), jnp.float32),
        jax.ShapeDtypeStruct((24, DIM), jnp.float32),
    ],
    scratch_types=[
        pltpu.VMEM((HIST, B_PER_NW), jnp.int32),
        pltpu.VMEM((B_PER_NW, DIM), jnp.float32),
        pltpu.VMEM((B_PER_NW, DIM), jnp.float32),
        pltpu.VMEM((B_PER_NW, DIM), jnp.float32),
        pltpu.VMEM((B_PER_NW, DIM), jnp.float32),
        pltpu.SemaphoreType.DMA,
        pltpu.SemaphoreType.DMA,
        pltpu.SemaphoreType.DMA,
        pltpu.SemaphoreType.DMA,
        pltpu.SemaphoreType.DMA,
        pltpu.SemaphoreType.DMA,
        pltpu.SemaphoreType.DMA,
        pltpu.SemaphoreType.DMA,
        pltpu.SemaphoreType.DMA,
    ],
    compiler_params=pltpu.CompilerParams(use_tc_tiling_on_sc=False),
)
def _gather_kernel(idx_hbm, loc_tab, ts_tab, loc_out, ts_out,
                   idx_v, buf0, buf1, buf2, buf3,
                   gsem0, gsem1, gsem2, gsem3,
                   wsem0, wsem1, wsem2, wsem3, isem):
    wid = lax.axis_index("s") * NC + lax.axis_index("c")
    hw = B_PER_NW // 2
    q0 = wid * hw

    for h in range(HIST):
        pltpu.async_copy(
            idx_hbm.at[pl.ds(h * BATCH + q0, hw)],
            idx_v.at[h, pl.ds(0, hw)], isem).wait()
        pltpu.async_copy(
            idx_hbm.at[pl.ds(h * BATCH + BATCH // 2 + q0, hw)],
            idx_v.at[h, pl.ds(hw, hw)], isem).wait()

    bufs = (buf0, buf1, buf2, buf3)
    gsems = (gsem0, gsem1, gsem2, gsem3)
    wsems = (wsem0, wsem1, wsem2, wsem3)
    NBUF = 4

    def _write(h, buf, sem):
        return (
            pltpu.async_copy(
                buf.at[pl.ds(0, hw)],
                loc_out.at[h, pl.ds(q0, hw), pl.ds(0, DIM)], sem),
            pltpu.async_copy(
                buf.at[pl.ds(hw, hw)],
                loc_out.at[h, pl.ds(q0, hw), pl.ds(DIM, DIM)], sem),
        )

    reads = [None] * NBUF
    writes = [None] * NBUF
    for t in range(HIST + NBUF - 1):
        if t < HIST:
            b = t % NBUF
            if writes[b] is not None:
                writes[b][0].wait()
                writes[b][1].wait()
            reads[b] = pltpu.async_copy(
                loc_tab.at[idx_v.at[t]], bufs[b], gsems[b])
        hp = t - (NBUF - 1)
        if 0 <= hp < HIST:
            pb = hp % NBUF
            reads[pb].wait()
            writes[pb] = _write(hp, bufs[pb], wsems[pb])
    for k in range(NBUF):
        if writes[k] is not None:
            writes[k][0].wait()
            writes[k][1].wait()

    @pl.when(wid == 0)
    def _():
        pltpu.sync_copy(ts_tab, buf0.at[pl.ds(0, 24)])
        pltpu.sync_copy(buf0.at[pl.ds(0, 24)], ts_out)


def _copy_body(in_ref, out_ref):
    out_ref[...] = in_ref[...]


_COLS_PER_BLK = 6400
_user_copy_t = pl.pallas_call(
    _copy_body,
    grid=(NUM_USERS // _COLS_PER_BLK + 1,),
    in_specs=[pl.BlockSpec((DIM, _COLS_PER_BLK), lambda i: (0, i))],
    out_specs=pl.BlockSpec((DIM, _COLS_PER_BLK), lambda i: (0, i)),
    out_shape=jax.ShapeDtypeStruct((DIM, NUM_USERS), jnp.float32),
)


def _unpack_body(x_ref, y_ref):
    x = x_ref[0]
    y_ref[0] = jnp.concatenate([x[:, :DIM].T, x[:, DIM:].T], axis=1)


_unpack = pl.pallas_call(
    _unpack_body,
    grid=(HIST,),
    in_specs=[pl.BlockSpec((1, BATCH // 2, 2 * DIM), lambda h: (h, 0, 0))],
    out_specs=pl.BlockSpec((1, DIM, BATCH), lambda h: (h, 0, 0)),
    out_shape=jax.ShapeDtypeStruct((HIST, DIM, BATCH), jnp.float32),
)


def kernel(location_x, location_table, user_table, timeslot_table):
    idx_t = location_x.T.reshape(BATCH * HIST).astype(jnp.int32)
    loc_p, ts = _gather_kernel(idx_t, location_table, timeslot_table)
    loc = jnp.transpose(_unpack(loc_p), (2, 0, 1))
    user = _user_copy_t(user_table.T).T
    return loc, ts, user


# speedup vs baseline: 1.1565x; 1.1565x over previous
import functools

import jax
import jax.numpy as jnp
from jax import lax
from jax.experimental import pallas as pl
from jax.experimental.pallas import tpu as pltpu
from jax.experimental.pallas import tpu_sc as plsc

NUM_LOCATIONS = 100000
NUM_USERS = 100000
DIM = 64
BATCH = 4096
HIST = 50

NC, NS = 2, 16
NW = NC * NS
B_PER_NW = BATCH // NW

_mesh = plsc.VectorSubcoreMesh(core_axis_name="c", subcore_axis_name="s")


@functools.partial(
    pl.kernel,
    mesh=_mesh,
    out_type=[
        jax.ShapeDtypeStruct((HIST, BATCH // 2, 2 * DIM), jnp.float32),
        jax.ShapeDtypeStruct((24, DIM), jnp.float32),
    ],
    scratch_types=[
        pltpu.VMEM((HIST, B_PER_NW), jnp.int32),
        pltpu.VMEM((B_PER_NW, DIM), jnp.float32),
        pltpu.VMEM((B_PER_NW, DIM), jnp.float32),
        pltpu.VMEM((B_PER_NW, DIM), jnp.float32),
        pltpu.VMEM((B_PER_NW, DIM), jnp.float32),
        pltpu.SemaphoreType.DMA,
        pltpu.SemaphoreType.DMA,
        pltpu.SemaphoreType.DMA,
        pltpu.SemaphoreType.DMA,
        pltpu.SemaphoreType.DMA,
        pltpu.SemaphoreType.DMA,
        pltpu.SemaphoreType.DMA,
        pltpu.SemaphoreType.DMA,
        pltpu.SemaphoreType.DMA,
    ],
    compiler_params=pltpu.CompilerParams(use_tc_tiling_on_sc=False),
)
def _gather_kernel(idx_hbm, loc_tab, ts_tab, loc_out, ts_out,
                   idx_v, buf0, buf1, buf2, buf3,
                   gsem0, gsem1, gsem2, gsem3,
                   wsem0, wsem1, wsem2, wsem3, isem):
    wid = lax.axis_index("s") * NC + lax.axis_index("c")
    c0 = wid * B_PER_NW
    qq = (wid % (NW // 2)) * B_PER_NW
    d0 = (wid // (NW // 2)) * DIM

    K = 8
    for h0 in range(0, HIST, K):
        hh = [pltpu.async_copy(
                  idx_hbm.at[pl.ds(h * BATCH + c0, B_PER_NW)],
                  idx_v.at[h], isem)
              for h in range(h0, min(h0 + K, HIST))]
        for hnd in hh:
            hnd.wait()

    bufs = (buf0, buf1, buf2, buf3)
    gsems = (gsem0, gsem1, gsem2, gsem3)
    wsems = (wsem0, wsem1, wsem2, wsem3)
    NBUF = 4

    def _write(h, buf, sem):
        return pltpu.async_copy(
            buf, loc_out.at[h, pl.ds(qq, B_PER_NW), pl.ds(d0, DIM)], sem)

    reads = [None] * NBUF
    writes = [None] * NBUF
    for t in range(HIST + NBUF - 1):
        if t < HIST:
            b = t % NBUF
            if writes[b] is not None:
                writes[b].wait()
            reads[b] = pltpu.async_copy(
                loc_tab.at[idx_v.at[t]], bufs[b], gsems[b])
        hp = t - (NBUF - 1)
        if 0 <= hp < HIST:
            pb = hp % NBUF
            reads[pb].wait()
            writes[pb] = _write(hp, bufs[pb], wsems[pb])
    for k in range(NBUF):
        if writes[k] is not None:
            writes[k].wait()

    @pl.when(wid == 0)
    def _():
        pltpu.sync_copy(ts_tab, buf0.at[pl.ds(0, 24)])
        pltpu.sync_copy(buf0.at[pl.ds(0, 24)], ts_out)


def _copy_body(in_ref, out_ref):
    out_ref[...] = in_ref[...]


_COLS_PER_BLK = 6400
_user_copy_t = pl.pallas_call(
    _copy_body,
    grid=(NUM_USERS // _COLS_PER_BLK + 1,),
    in_specs=[pl.BlockSpec((DIM, _COLS_PER_BLK), lambda i: (0, i))],
    out_specs=pl.BlockSpec((DIM, _COLS_PER_BLK), lambda i: (0, i)),
    out_shape=jax.ShapeDtypeStruct((DIM, NUM_USERS), jnp.float32),
)


def _unpack_body(x_ref, y_ref):
    x = x_ref[0]
    y_ref[0] = jnp.concatenate([x[:, :DIM].T, x[:, DIM:].T], axis=1)


_unpack = pl.pallas_call(
    _unpack_body,
    grid=(HIST,),
    in_specs=[pl.BlockSpec((1, BATCH // 2, 2 * DIM), lambda h: (h, 0, 0))],
    out_specs=pl.BlockSpec((1, DIM, BATCH), lambda h: (h, 0, 0)),
    out_shape=jax.ShapeDtypeStruct((HIST, DIM, BATCH), jnp.float32),
)


def kernel(location_x, location_table, user_table, timeslot_table):
    idx_t = location_x.T.reshape(BATCH * HIST).astype(jnp.int32)
    loc_p, ts = _gather_kernel(idx_t, location_table, timeslot_table)
    loc = jnp.transpose(_unpack(loc_p), (2, 0, 1))
    user = _user_copy_t(user_table.T).T
    return loc, ts, user
